# baseline (speedup 1.0000x reference)
import functools

import jax
import jax.numpy as jnp
from jax import lax
from jax.experimental import pallas as pl
from jax.experimental.pallas import tpu as pltpu
from jax.experimental.pallas import tpu_sc as plsc

VOCAB = 1000000
D = 64
MAXLEN = 200
VDIM = 128
B = 1024
T = 200
BT = B * T

NC = 2
NS = 16
NW = NC * NS
PER_W = BT // NW
STREAM = 128
N_STREAMS = PER_W // STREAM
GROUP_STREAMS = 5
GROUP = GROUP_STREAMS * STREAM
N_GROUPS = PER_W // GROUP


def _sc_gather_body(table_hbm, idx_hbm, out_hbm, idx_v, rows_v, sem):
    wid = lax.axis_index("s") * NC + lax.axis_index("c")
    base = wid * PER_W
    pltpu.sync_copy(idx_hbm.at[wid], idx_v)

    @pl.loop(0, N_GROUPS)
    def _group(g):
        copies = []
        for j in range(GROUP_STREAMS):
            c = pltpu.async_copy(
                table_hbm.at[idx_v.at[g * GROUP_STREAMS + j]],
                rows_v.at[pl.ds(j * STREAM, STREAM)],
                sem,
            )
            copies.append(c)
        for c in copies:
            c.wait()
        pltpu.sync_copy(rows_v, out_hbm.at[pl.ds(base + g * GROUP, GROUP)])


@functools.partial(jax.jit, donate_argnums=())
def _sc_gather(table, idx):
    mesh = plsc.VectorSubcoreMesh(core_axis_name="c", subcore_axis_name="s")
    return pl.kernel(
        _sc_gather_body,
        out_type=jax.ShapeDtypeStruct((BT, D), jnp.float32),
        mesh=mesh,
        scratch_types=[
            pltpu.VMEM((N_STREAMS, STREAM), jnp.int32),
            pltpu.VMEM((GROUP, D), jnp.float32),
            pltpu.SemaphoreType.DMA,
        ],
        compiler_params=pltpu.CompilerParams(use_tc_tiling_on_sc=False),
    )(table, idx)


BLK = 1600


def _tc_body(g_ref, vis_ref, pos_ref, w_ref, gamma_ref, beta_ref, out_ref):
    x = g_ref[...] + pos_ref[...]
    x = x + jnp.dot(vis_ref[...], w_ref[...], preferred_element_type=jnp.float32)
    mean = jnp.mean(x, axis=-1, keepdims=True)
    xc = x - mean
    var = jnp.mean(xc * xc, axis=-1, keepdims=True)
    out_ref[...] = xc * lax.rsqrt(var + 1e-6) * gamma_ref[...] + beta_ref[...]


def _tc_fused(gathered, vis, pos_tiled, w_t, gamma, beta):
    grid = (BT // BLK,)
    return pl.pallas_call(
        _tc_body,
        grid=grid,
        in_specs=[
            pl.BlockSpec((BLK, D), lambda i: (i, 0)),
            pl.BlockSpec((BLK, VDIM), lambda i: (i, 0)),
            pl.BlockSpec((BLK, D), lambda i: (0, 0)),
            pl.BlockSpec((VDIM, D), lambda i: (0, 0)),
            pl.BlockSpec((1, D), lambda i: (0, 0)),
            pl.BlockSpec((1, D), lambda i: (0, 0)),
        ],
        out_specs=pl.BlockSpec((BLK, D), lambda i: (i, 0)),
        out_shape=jax.ShapeDtypeStruct((BT, D), jnp.float32),
    )(gathered, vis, pos_tiled, w_t, gamma, beta)


def kernel(seq, visual_features, token_table, pos_table, W_visual, ln_gamma, ln_beta):
    idx = seq.astype(jnp.int32).reshape(NW, N_STREAMS, STREAM)
    gathered = _sc_gather(token_table, idx)
    vis = visual_features.reshape(BT, VDIM)
    pos_tiled = jnp.tile(pos_table, (BLK // T, 1))
    out = _tc_fused(
        gathered,
        vis,
        pos_tiled,
        W_visual.T,
        ln_gamma.reshape(1, D),
        ln_beta.reshape(1, D),
    )
    return out.reshape(B, T, D)
